# Initial kernel scaffold; baseline (speedup 1.0000x reference)
import jax
import jax.numpy as jnp
from jax.experimental import pallas as pl


def kernel(seq_feat, msa_feat, ln_gamma, ln_beta, gate_w, gate_b):
    raise NotImplementedError("write your pallas kernel here")


# speedup vs baseline: 2.5495x; 2.5495x over previous
import functools

import jax
import jax.numpy as jnp
from jax.experimental import pallas as pl
from jax.experimental.pallas import tpu as pltpu

_LN_EPS = 1e-5
_ROWS_PER_BLOCK = 1024


def _fused_body(seq_ref, msa_ref, gw_ref, gamma_ref, beta_ref, gb_ref, out_ref):
    g = gamma_ref[...]
    bt = beta_ref[...]
    w = gw_ref[...]
    dws = g[0:1, :] * (w[0:1, :] - w[2:3, :])
    dwm = g[1:2, :] * (w[1:2, :] - w[3:4, :])
    da = jnp.sum(dws, axis=-1, keepdims=True) + jnp.sum(dwm, axis=-1, keepdims=True)
    dcb = (jnp.sum(bt[0:1, :] * (w[0:1, :] - w[2:3, :]), axis=-1, keepdims=True)
           + jnp.sum(bt[1:2, :] * (w[1:2, :] - w[3:4, :]), axis=-1, keepdims=True)
           + (gb_ref[0:1, 0:1] - gb_ref[0:1, 1:2]))

    s = seq_ref[...]
    m = msa_ref[...]
    n_inv = 1.0 / (2.0 * s.shape[-1])
    row_sum = (jnp.sum(s, axis=-1, keepdims=True)
               + jnp.sum(m, axis=-1, keepdims=True))
    row_sq = (jnp.sum(s * s, axis=-1, keepdims=True)
              + jnp.sum(m * m, axis=-1, keepdims=True))
    mean = row_sum * n_inv
    var = row_sq * n_inv - mean * mean
    rstd = jax.lax.rsqrt(var + _LN_EPS)
    t = (jnp.sum(s * dws, axis=-1, keepdims=True)
         + jnp.sum(m * dwm, axis=-1, keepdims=True))
    dl = (t - mean * da) * rstd + dcb
    w0 = jax.nn.sigmoid(dl)
    out_ref[...] = m + w0 * (s - m)


@jax.jit
def kernel(seq_feat, msa_feat, ln_gamma, ln_beta, gate_w, gate_b):
    B, S, D = seq_feat.shape
    rows = B * S
    seq2 = seq_feat.reshape(rows, D)
    msa2 = msa_feat.reshape(rows, D)
    gamma2 = ln_gamma.reshape(2, D)
    beta2 = ln_beta.reshape(2, D)
    gw = gate_w.reshape(4, D)
    gb = gate_b.reshape(1, 2)

    nblk = rows // _ROWS_PER_BLOCK
    row_spec = pl.BlockSpec((_ROWS_PER_BLOCK, D), lambda i: (i, 0))
    full = lambda shape: pl.BlockSpec(shape, lambda i: (0,) * len(shape))

    out = pl.pallas_call(
        _fused_body,
        out_shape=jax.ShapeDtypeStruct((rows, D), seq_feat.dtype),
        grid=(nblk,),
        in_specs=[
            row_spec,
            row_spec,
            full((4, D)),
            full((2, D)),
            full((2, D)),
            full((1, 2)),
        ],
        out_specs=row_spec,
        compiler_params=pltpu.CompilerParams(
            dimension_semantics=("parallel",),
        ),
        name="attention_fusion",
    )(seq2, msa2, gw, gamma2, beta2, gb)
    return out.reshape(B, S, D)


# speedup vs baseline: 2.6402x; 1.0356x over previous
import functools

import jax
import jax.numpy as jnp
from jax.experimental import pallas as pl
from jax.experimental.pallas import tpu as pltpu

_LN_EPS = 1e-5
_ROWS_PER_BLOCK = 2048


def _fused_body(seq_ref, msa_ref, gw_ref, gamma_ref, beta_ref, gb_ref, out_ref):
    g = gamma_ref[...]
    bt = beta_ref[...]
    w = gw_ref[...]
    dws = g[0:1, :] * (w[0:1, :] - w[2:3, :])
    dwm = g[1:2, :] * (w[1:2, :] - w[3:4, :])
    da = jnp.sum(dws, axis=-1, keepdims=True) + jnp.sum(dwm, axis=-1, keepdims=True)
    dcb = (jnp.sum(bt[0:1, :] * (w[0:1, :] - w[2:3, :]), axis=-1, keepdims=True)
           + jnp.sum(bt[1:2, :] * (w[1:2, :] - w[3:4, :]), axis=-1, keepdims=True)
           + (gb_ref[0:1, 0:1] - gb_ref[0:1, 1:2]))

    s = seq_ref[...]
    m = msa_ref[...]
    n_inv = 1.0 / (2.0 * s.shape[-1])
    row_sum = (jnp.sum(s, axis=-1, keepdims=True)
               + jnp.sum(m, axis=-1, keepdims=True))
    row_sq = (jnp.sum(s * s, axis=-1, keepdims=True)
              + jnp.sum(m * m, axis=-1, keepdims=True))
    mean = row_sum * n_inv
    var = row_sq * n_inv - mean * mean
    rstd = jax.lax.rsqrt(var + _LN_EPS)
    t = (jnp.sum(s * dws, axis=-1, keepdims=True)
         + jnp.sum(m * dwm, axis=-1, keepdims=True))
    dl = (t - mean * da) * rstd + dcb
    w0 = jax.nn.sigmoid(dl)
    out_ref[...] = m + w0 * (s - m)


@jax.jit
def kernel(seq_feat, msa_feat, ln_gamma, ln_beta, gate_w, gate_b):
    B, S, D = seq_feat.shape
    rows = B * S
    seq2 = seq_feat.reshape(rows, D)
    msa2 = msa_feat.reshape(rows, D)
    gamma2 = ln_gamma.reshape(2, D)
    beta2 = ln_beta.reshape(2, D)
    gw = gate_w.reshape(4, D)
    gb = gate_b.reshape(1, 2)

    nblk = rows // _ROWS_PER_BLOCK
    row_spec = pl.BlockSpec((_ROWS_PER_BLOCK, D), lambda i: (i, 0))
    full = lambda shape: pl.BlockSpec(shape, lambda i: (0,) * len(shape))

    out = pl.pallas_call(
        _fused_body,
        out_shape=jax.ShapeDtypeStruct((rows, D), seq_feat.dtype),
        grid=(nblk,),
        in_specs=[
            row_spec,
            row_spec,
            full((4, D)),
            full((2, D)),
            full((2, D)),
            full((1, 2)),
        ],
        out_specs=row_spec,
        compiler_params=pltpu.CompilerParams(
            dimension_semantics=("parallel",),
        ),
        name="attention_fusion",
    )(seq2, msa2, gw, gamma2, beta2, gb)
    return out.reshape(B, S, D)
